# baseline (speedup 1.0000x reference)
import jax
import jax.numpy as jnp
from jax import lax
from jax.experimental import pallas as pl
from jax.experimental.pallas import tpu as pltpu
from jax.experimental.pallas import tpu_sc as plsc

BATCH = 4096
HIST = 50
EMB_DIM = 64

NC = 2
NS = 16
NW = NC * NS

B_PER_W = BATCH // NW
CB = 16
NCHUNK = B_PER_W // CB


def _body(x_hbm, table_hbm, out_hbm, idx_v, rows_v, sem_g, sem_o):
    wid = lax.axis_index("s") * NC + lax.axis_index("c")
    base_w = wid * B_PER_W

    pltpu.sync_copy(x_hbm.at[pl.ds(base_w, B_PER_W)], idx_v)

    def store_ref(g):
        return out_hbm.at[pl.ds(base_w + g * CB, CB)]

    def chunk(g, carry):
        b = lax.rem(g, 2)
        buf = rows_v.at[b]

        @pl.when(g >= 2)
        def _():
            pltpu.make_async_copy(buf, store_ref(g - 2), sem_o).wait()

        copies = [
            pltpu.async_copy(
                table_hbm.at[idx_v.at[g * CB + j]],
                buf.at[j],
                sem_g,
            )
            for j in range(CB)
        ]
        for c in copies:
            c.wait()

        pltpu.make_async_copy(buf, store_ref(g), sem_o).start()
        return carry

    lax.fori_loop(0, NCHUNK, chunk, 0)
    pltpu.make_async_copy(rows_v.at[0], store_ref(NCHUNK - 2), sem_o).wait()
    pltpu.make_async_copy(rows_v.at[1], store_ref(NCHUNK - 1), sem_o).wait()


@jax.jit
def _gather(x, table):
    run = pl.kernel(
        _body,
        out_type=jax.ShapeDtypeStruct((BATCH, HIST, EMB_DIM), jnp.float32),
        mesh=plsc.VectorSubcoreMesh(core_axis_name="c", subcore_axis_name="s"),
        compiler_params=pltpu.CompilerParams(use_tc_tiling_on_sc=False),
        scratch_types=[
            pltpu.VMEM((B_PER_W, HIST), jnp.int32),
            pltpu.VMEM((2, CB, HIST, EMB_DIM), jnp.float32),
            pltpu.SemaphoreType.DMA,
            pltpu.SemaphoreType.DMA,
        ],
    )
    return run(x, table)


def kernel(x, table):
    return _gather(x.astype(jnp.int32), table)


# speedup vs baseline: 1.3969x; 1.3969x over previous
import jax
import jax.numpy as jnp
from jax import lax
from jax.experimental import pallas as pl
from jax.experimental.pallas import tpu as pltpu
from jax.experimental.pallas import tpu_sc as plsc

BATCH = 4096
HIST = 50
EMB_DIM = 64
PAD_DIM = 128

NC = 2
NS = 16
NW = NC * NS

BW = BATCH // NW
HC = 2
NCH = HIST // HC
LANES = 16


def _body(xt_hbm, table_hbm, out_hbm, idx_v, gbuf, tbuf, sem_g, sem_o):
    wid = lax.axis_index("s") * NC + lax.axis_index("c")
    b0 = wid * BW

    pltpu.sync_copy(xt_hbm.at[:, pl.ds(b0, BW)], idx_v)

    lane = jnp.arange(LANES, dtype=jnp.int32)
    diags = [(lane + k) & (LANES - 1) for k in range(LANES)]

    def gather_start(g, p):
        for hh in range(HC):
            pltpu.make_async_copy(
                table_hbm.at[idx_v.at[g * HC + hh]],
                gbuf.at[p, hh],
                sem_g,
            ).start()

    def gather_wait(g, p):
        for hh in range(HC):
            pltpu.make_async_copy(
                table_hbm.at[idx_v.at[g * HC + hh]],
                gbuf.at[p, hh],
                sem_g,
            ).wait()

    def store_copy(g, p):
        return pltpu.make_async_copy(
            tbuf.at[p],
            out_hbm.at[pl.ds(g * HC, HC), :, pl.ds(b0, BW)],
            sem_o,
        )

    def transpose(p):
        for hh in range(HC):
            rows = gbuf.at[p, hh]
            tp = tbuf.at[p, hh]

            def block(bi, carry):
                d0 = (bi % (EMB_DIM // LANES)) * LANES
                bb = (bi // (EMB_DIM // LANES)) * LANES
                row = bb + lane
                for k in range(LANES):
                    col = d0 + diags[k]
                    v = plsc.load_gather(rows, [row, col])
                    plsc.store_scatter(tp, [col, row], v)
                return carry

            lax.fori_loop(0, (EMB_DIM // LANES) * (BW // LANES), block, 0)

    gather_start(0, 0)

    def chunk(g, carry):
        p = lax.rem(g, 2)
        gather_wait(g, p)

        @pl.when(g + 1 < NCH)
        def _():
            gather_start(g + 1, 1 - p)

        @pl.when(g >= 2)
        def _():
            store_copy(g - 2, p).wait()

        transpose(p)
        store_copy(g, p).start()
        return carry

    lax.fori_loop(0, NCH, chunk, 0)
    store_copy(NCH - 2, (NCH - 2) % 2).wait()
    store_copy(NCH - 1, (NCH - 1) % 2).wait()


@jax.jit
def _gather(xt, table_p):
    run = pl.kernel(
        _body,
        out_type=jax.ShapeDtypeStruct((HIST, EMB_DIM, BATCH), jnp.float32),
        mesh=plsc.VectorSubcoreMesh(core_axis_name="c", subcore_axis_name="s"),
        compiler_params=pltpu.CompilerParams(
            use_tc_tiling_on_sc=True, needs_layout_passes=False
        ),
        scratch_types=[
            pltpu.VMEM((HIST, BW), jnp.int32),
            pltpu.VMEM((2, HC, BW, PAD_DIM), jnp.float32),
            pltpu.VMEM((2, HC, EMB_DIM, BW), jnp.float32),
            pltpu.SemaphoreType.DMA,
            pltpu.SemaphoreType.DMA,
        ],
    )
    return run(xt, table_p)


def kernel(x, table):
    xt = x.astype(jnp.int32).T
    table_p = jnp.pad(table, ((0, 0), (0, PAD_DIM - EMB_DIM)))
    out_t = _gather(xt, table_p)
    return jnp.transpose(out_t, (2, 0, 1))


# speedup vs baseline: 1.4168x; 1.0142x over previous
import jax
import jax.numpy as jnp
from jax import lax
from jax.experimental import pallas as pl
from jax.experimental.pallas import tpu as pltpu
from jax.experimental.pallas import tpu_sc as plsc

BATCH = 4096
HIST = 50
EMB_DIM = 64
PAD_DIM = 128
PAIRS = 50001

NC = 2
NS = 16
NW = NC * NS

BW = BATCH // NW
HC = 2
NCH = HIST // HC
LANES = 16
DBLK = EMB_DIM // LANES
BBLK = BW // LANES


def _body(xt_hbm, table_hbm, out_hbm, idx_v, pidx_v, gbuf, tbuf, sem_g, sem_o):
    wid = lax.axis_index("s") * NC + lax.axis_index("c")
    b0 = wid * BW

    pltpu.sync_copy(xt_hbm.at[:, pl.ds(b0, BW)], idx_v)

    lane = jnp.arange(LANES, dtype=jnp.int32)
    diags = [(lane + k) & (LANES - 1) for k in range(LANES)]

    def to_pairs(h, carry):
        for c in range(BBLK):
            pidx_v[h, pl.ds(c * LANES, LANES)] = (
                idx_v[h, pl.ds(c * LANES, LANES)] >> 1
            )
        return carry

    lax.fori_loop(0, HIST, to_pairs, 0)

    def gather_copy(g, p, hh):
        return pltpu.make_async_copy(
            table_hbm.at[pidx_v.at[g * HC + hh]],
            gbuf.at[p, hh],
            sem_g,
        )

    def gather_start(g, p):
        for hh in range(HC):
            gather_copy(g, p, hh).start()

    def gather_wait(g, p):
        for hh in range(HC):
            gather_copy(g, p, hh).wait()

    def store_copy(g, p):
        return pltpu.make_async_copy(
            tbuf.at[p],
            out_hbm.at[pl.ds(g * HC, HC), :, pl.ds(b0, BW)],
            sem_o,
        )

    def transpose(g, p):
        for hh in range(HC):
            h_abs = g * HC + hh
            rows = gbuf.at[p, hh]
            tp = tbuf.at[p, hh]

            @plsc.parallel_loop(0, DBLK * BBLK, unroll=2)
            def _(bi):
                d0 = (bi % DBLK) * LANES
                bb = (bi // DBLK) * LANES
                row = bb + lane
                par = (idx_v[h_abs, pl.ds(bb, LANES)] & 1) << 6
                for k in range(LANES):
                    dcol = d0 + diags[k]
                    v = plsc.load_gather(rows, [row, dcol + par])
                    plsc.store_scatter(tp, [dcol, row], v)

    gather_start(0, 0)

    def chunk(g, carry):
        p = lax.rem(g, 2)
        gather_wait(g, p)

        @pl.when(g + 1 < NCH)
        def _():
            gather_start(g + 1, 1 - p)

        @pl.when(g >= 2)
        def _():
            store_copy(g - 2, p).wait()

        transpose(g, p)
        store_copy(g, p).start()
        return carry

    lax.fori_loop(0, NCH, chunk, 0)
    store_copy(NCH - 2, (NCH - 2) % 2).wait()
    store_copy(NCH - 1, (NCH - 1) % 2).wait()


@jax.jit
def _gather(xt, table_p):
    run = pl.kernel(
        _body,
        out_type=jax.ShapeDtypeStruct((HIST, EMB_DIM, BATCH), jnp.float32),
        mesh=plsc.VectorSubcoreMesh(core_axis_name="c", subcore_axis_name="s"),
        compiler_params=pltpu.CompilerParams(
            use_tc_tiling_on_sc=True, needs_layout_passes=False
        ),
        scratch_types=[
            pltpu.VMEM((HIST, BW), jnp.int32),
            pltpu.VMEM((HIST, BW), jnp.int32),
            pltpu.VMEM((2, HC, BW, PAD_DIM), jnp.float32),
            pltpu.VMEM((2, HC, EMB_DIM, BW), jnp.float32),
            pltpu.SemaphoreType.DMA,
            pltpu.SemaphoreType.DMA,
        ],
    )
    return run(xt, table_p)


def kernel(x, table):
    xt = x.astype(jnp.int32).T
    table_p = jnp.pad(table, ((0, 1), (0, 0))).reshape(PAIRS, PAD_DIM)
    out_t = _gather(xt, table_p)
    return jnp.transpose(out_t, (2, 0, 1))


# speedup vs baseline: 1.8434x; 1.3012x over previous
import jax
import jax.numpy as jnp
from jax import lax
from jax.experimental import pallas as pl
from jax.experimental.pallas import tpu as pltpu
from jax.experimental.pallas import tpu_sc as plsc

BATCH = 4096
HIST = 50
EMB_DIM = 64
PAD_DIM = 128

NC = 2
NS = 16
NW = NC * NS

BW = BATCH // NW
HC = 2
NCH = HIST // HC
LANES = 16
DBLK = EMB_DIM // LANES
BBLK = BW // LANES


def _body(xt_hbm, table_hbm, out_hbm, idx_v, gbuf, tbuf, sem_g, sem_o):
    wid = lax.axis_index("s") * NC + lax.axis_index("c")
    b0 = wid * BW

    pltpu.sync_copy(xt_hbm.at[:, pl.ds(b0, BW)], idx_v)

    lane = jnp.arange(LANES, dtype=jnp.int32)
    diags = [(lane + k) & (LANES - 1) for k in range(LANES)]

    def gather_copy(g, p, hh):
        return pltpu.make_async_copy(
            table_hbm.at[idx_v.at[g * HC + hh]],
            gbuf.at[p, hh],
            sem_g,
        )

    def gather_start(g, p):
        for hh in range(HC):
            gather_copy(g, p, hh).start()

    def gather_wait(g, p):
        for hh in range(HC):
            gather_copy(g, p, hh).wait()

    def store_copy(g, p):
        return pltpu.make_async_copy(
            tbuf.at[p],
            out_hbm.at[pl.ds(g * HC, HC), :, pl.ds(b0, BW)],
            sem_o,
        )

    def transpose(p):
        for hh in range(HC):
            rows = gbuf.at[p, hh]
            tp = tbuf.at[p, hh]

            @plsc.parallel_loop(0, DBLK * BBLK, unroll=2)
            def _(bi):
                d0 = (bi % DBLK) * LANES
                bb = (bi // DBLK) * LANES
                row = bb + lane
                for k in range(LANES):
                    dcol = d0 + diags[k]
                    v = plsc.load_gather(rows, [row, dcol])
                    plsc.store_scatter(tp, [dcol, row], v)

    gather_start(0, 0)

    def chunk(g, carry):
        p = lax.rem(g, 2)
        gather_wait(g, p)

        @pl.when(g + 1 < NCH)
        def _():
            gather_start(g + 1, 1 - p)

        @pl.when(g >= 2)
        def _():
            store_copy(g - 2, p).wait()

        transpose(p)
        store_copy(g, p).start()
        return carry

    lax.fori_loop(0, NCH, chunk, 0)
    store_copy(NCH - 2, (NCH - 2) % 2).wait()
    store_copy(NCH - 1, (NCH - 1) % 2).wait()


@jax.jit
def _gather(xt, table_p):
    run = pl.kernel(
        _body,
        out_type=jax.ShapeDtypeStruct((HIST, EMB_DIM, BATCH), jnp.float32),
        mesh=plsc.VectorSubcoreMesh(core_axis_name="c", subcore_axis_name="s"),
        compiler_params=pltpu.CompilerParams(
            use_tc_tiling_on_sc=True, needs_layout_passes=False
        ),
        scratch_types=[
            pltpu.VMEM((HIST, BW), jnp.int32),
            pltpu.VMEM((2, HC, BW, PAD_DIM), jnp.float32),
            pltpu.VMEM((2, HC, EMB_DIM, BW), jnp.float32),
            pltpu.SemaphoreType.DMA,
            pltpu.SemaphoreType.DMA,
        ],
    )
    return run(xt, table_p)


def kernel(x, table):
    xt = x.astype(jnp.int32).T
    table_p = jnp.pad(table, ((0, 0), (0, PAD_DIM - EMB_DIM)))
    out_t = _gather(xt, table_p)
    return jnp.transpose(out_t, (2, 0, 1))
